# baseline (speedup 1.0000x reference)
import functools

import jax
import jax.numpy as jnp
from jax import lax
from jax.experimental import pallas as pl
from jax.experimental.pallas import tpu as pltpu
from jax.experimental.pallas import tpu_sc as plsc

N = 10000
E = 320000
NNZ = 500000
IN_C = 128
OUT_C = 128
H = 64
NC = 2
NS = 16
L = 16
NB = 3

CHUNK = 128
NNZ_CT = 249
NNZ_PAD = NNZ_CT * NS * CHUNK
E_CT = 162
E_PAD = E_CT * NS * CHUNK
ACC_PAD = N * H + 512
ZSLICE = ACC_PAD // NS
OSLICE = N * H // NS
RPT = N // NS

_mesh = plsc.VectorSubcoreMesh(core_axis_name="c", subcore_axis_name="s")
_sc_params = pltpu.CompilerParams(use_tc_tiling_on_sc=False,
                                  needs_layout_passes=False)


@functools.partial(
    pl.kernel,
    out_type=jax.ShapeDtypeStruct((NC * N * H,), jnp.float32),
    mesh=_mesh,
    scratch_types=(
        [pltpu.VMEM((CHUNK,), jnp.int32) for _ in range(NB)]
        + [pltpu.VMEM((CHUNK,), jnp.int32) for _ in range(NB)]
        + [pltpu.VMEM((CHUNK,), jnp.float32) for _ in range(NB)]
        + [pltpu.VMEM((CHUNK,), jnp.int32) for _ in range(NB)]
        + [pltpu.VMEM((ZSLICE,), jnp.float32)]
        + [pltpu.VMEM_SHARED((ACC_PAD,), jnp.float32)]
        + [pltpu.SemaphoreType.DMA for _ in range(2 * NB)]
    ),
    compiler_params=_sc_params,
)
def _fbuild(rows_hbm, cols_hbm, vals_hbm, f_hbm, *refs):
    rowv = refs[0:NB]
    colv = refs[NB:2 * NB]
    valv = refs[2 * NB:3 * NB]
    idxv = refs[3 * NB:4 * NB]
    stage = refs[4 * NB]
    acc = refs[4 * NB + 1]
    isem = refs[4 * NB + 2:4 * NB + 2 + NB]
    ssem = refs[4 * NB + 2 + NB:4 * NB + 2 + 2 * NB]

    c = lax.axis_index("c")
    s = lax.axis_index("s")

    @pl.loop(0, ZSLICE, step=L)
    def _(i):
        stage[pl.ds(i, L)] = jnp.zeros((L,), jnp.float32)

    pltpu.sync_copy(stage, acc.at[pl.ds(s * ZSLICE, ZSLICE)])
    plsc.subcore_barrier()
    col_off = c * H

    def idx_start(i, b):
        base = (s + i * NS) * CHUNK
        pltpu.async_copy(rows_hbm.at[pl.ds(base, CHUNK)], rowv[b], isem[b])
        pltpu.async_copy(cols_hbm.at[pl.ds(base, CHUNK)], colv[b], isem[b])
        pltpu.async_copy(vals_hbm.at[pl.ds(base, CHUNK)], valv[b], isem[b])

    def idx_wait(b):
        pltpu.make_async_copy(rows_hbm.at[pl.ds(0, CHUNK)], rowv[b], isem[b]).wait()
        pltpu.make_async_copy(cols_hbm.at[pl.ds(0, CHUNK)], colv[b], isem[b]).wait()
        pltpu.make_async_copy(vals_hbm.at[pl.ds(0, CHUNK)], valv[b], isem[b]).wait()

    def compute(b):
        for j in range(CHUNK // L):
            sl = pl.ds(j * L, L)
            r = rowv[b][sl]
            cc = colv[b][sl] - col_off
            ok = (cc >= 0) & (cc < H)
            idxv[b][sl] = jnp.where(ok, r * H + cc, N * H)

    def scatter_start(b):
        pltpu.async_copy(valv[b], acc.at[idxv[b]], ssem[b], add=True)

    def scatter_wait(b):
        pltpu.make_async_copy(valv[b], acc.at[idxv[b]], ssem[b]).wait()

    def process(i, b, first, last):
        fb = (b + 2) % NB
        idx_wait(b)
        compute(b)
        scatter_start(b)
        if not first:
            scatter_wait(fb)
        if not last:
            idx_start(jnp.minimum(i + 2, NNZ_CT - 1), fb)

    idx_start(0, 0)
    idx_start(1, 1)
    process(jnp.int32(0), 0, first=True, last=False)
    process(jnp.int32(1), 1, first=False, last=False)
    process(jnp.int32(2), 2, first=False, last=False)

    @pl.loop(1, NNZ_CT // NB - 1)
    def _(t):
        i = t * NB
        process(i, 0, first=False, last=False)
        process(i + 1, 1, first=False, last=False)
        process(i + 2, 2, first=False, last=False)

    process(jnp.int32(NNZ_CT - 3), 0, first=False, last=False)
    process(jnp.int32(NNZ_CT - 2), 1, first=False, last=False)
    process(jnp.int32(NNZ_CT - 1), 2, first=False, last=True)
    scatter_wait(2)
    idx_wait(NNZ_CT % NB)

    plsc.subcore_barrier()
    pltpu.sync_copy(acc.at[pl.ds(s * OSLICE, OSLICE)], stage.at[pl.ds(0, OSLICE)])
    pltpu.sync_copy(stage.at[pl.ds(0, OSLICE)],
                    f_hbm.at[pl.ds(c * N * H + s * OSLICE, OSLICE)])


def _dense_body(f_ref, w_ref, b_ref, o_ref):
    f = f_ref[...]
    o = jnp.dot(f[0], w_ref[0], preferred_element_type=jnp.float32)
    o = o + jnp.dot(f[1], w_ref[1], preferred_element_type=jnp.float32)
    o = jnp.maximum(o + b_ref[...], 0.0)
    o_ref[...] = jnp.stack([o[:, :H], o[:, H:]])


_BM = 2000


def _dense(f3, w3, bias):
    return pl.pallas_call(
        _dense_body,
        grid=(N // _BM,),
        in_specs=[
            pl.BlockSpec((NC, _BM, H), lambda i: (0, i, 0)),
            pl.BlockSpec((NC, H, OUT_C), lambda i: (0, 0, 0)),
            pl.BlockSpec((1, OUT_C), lambda i: (0, 0)),
        ],
        out_specs=pl.BlockSpec((NC, _BM, H), lambda i: (0, i, 0)),
        out_shape=jax.ShapeDtypeStruct((NC, N, H), jnp.float32),
    )(f3, w3, bias)


@functools.partial(
    pl.kernel,
    out_type=[jax.ShapeDtypeStruct((NC, N, H), jnp.float32),
              jax.ShapeDtypeStruct((NC, N, H), jnp.float32)],
    mesh=_mesh,
    scratch_types=(
        [pltpu.VMEM((CHUNK,), jnp.int32) for _ in range(NB)]
        + [pltpu.VMEM((CHUNK,), jnp.int32) for _ in range(NB)]
        + [pltpu.VMEM((CHUNK,), jnp.float32) for _ in range(NB)]
        + [pltpu.VMEM((CHUNK, H), jnp.float32) for _ in range(NB)]
        + [pltpu.VMEM((RPT // 5, H), jnp.float32)]
        + [pltpu.VMEM_SHARED((N, H), jnp.float32)]
        + [pltpu.VMEM_SHARED((N, H), jnp.float32)]
        + [pltpu.SemaphoreType.DMA for _ in range(3 * NB)]
    ),
    compiler_params=_sc_params,
)
def _prop(adjr_hbm, adjc_hbm, adjv_hbm, base3_hbm, mid3_hbm, out3_hbm, *refs):
    rowv = refs[0:NB]
    colv = refs[NB:2 * NB]
    valv = refs[2 * NB:3 * NB]
    rbuf = refs[3 * NB:4 * NB]
    stage = refs[4 * NB]
    acc1 = refs[4 * NB + 1]
    acc2 = refs[4 * NB + 2]
    isem = refs[4 * NB + 3:4 * NB + 3 + NB]
    gsem = refs[4 * NB + 3 + NB:4 * NB + 3 + 2 * NB]
    ssem = refs[4 * NB + 3 + 2 * NB:4 * NB + 3 + 3 * NB]

    c = lax.axis_index("c")
    s = lax.axis_index("s")

    @pl.loop(0, RPT // 5)
    def _(k):
        for j in range(H // L):
            stage[k, pl.ds(j * L, L)] = jnp.zeros((L,), jnp.float32)

    for p in range(5):
        pltpu.sync_copy(stage, acc1.at[pl.ds(s * RPT + p * (RPT // 5), RPT // 5), :])
        pltpu.sync_copy(stage, acc2.at[pl.ds(s * RPT + p * (RPT // 5), RPT // 5), :])
    plsc.subcore_barrier()

    def idx_start(i, b):
        base = (s + i * NS) * CHUNK
        pltpu.async_copy(adjc_hbm.at[pl.ds(base, CHUNK)], colv[b], isem[b])
        pltpu.async_copy(adjr_hbm.at[pl.ds(base, CHUNK)], rowv[b], isem[b])
        pltpu.async_copy(adjv_hbm.at[pl.ds(base, CHUNK)], valv[b], isem[b])

    def idx_wait(b):
        pltpu.make_async_copy(adjc_hbm.at[pl.ds(0, CHUNK)], colv[b], isem[b]).wait()
        pltpu.make_async_copy(adjr_hbm.at[pl.ds(0, CHUNK)], rowv[b], isem[b]).wait()
        pltpu.make_async_copy(adjv_hbm.at[pl.ds(0, CHUNK)], valv[b], isem[b]).wait()

    def compute(b):
        @pl.loop(0, CHUNK, step=L)
        def _(k0):
            for e in range(L):
                v = plsc.load_gather(valv[b], [jnp.full((L,), k0 + e, jnp.int32)])
                for j in range(H // L):
                    sl = pl.ds(j * L, L)
                    rbuf[b][k0 + e, sl] = rbuf[b][k0 + e, sl] * v

    def hop(src, dst_acc):
        def gather_start(b):
            pltpu.async_copy(src.at[colv[b]], rbuf[b], gsem[b])

        def gather_wait(b):
            pltpu.make_async_copy(src.at[colv[b]], rbuf[b], gsem[b]).wait()

        def scatter_start(b):
            pltpu.async_copy(rbuf[b], dst_acc.at[rowv[b]], ssem[b], add=True)

        def scatter_wait(b):
            pltpu.make_async_copy(rbuf[b], dst_acc.at[rowv[b]], ssem[b]).wait()

        def process(i, b, first, last):
            nb = (b + 1) % NB
            fb = (b + 2) % NB
            gather_wait(b)
            compute(b)
            scatter_start(b)
            if not last:
                idx_wait(nb)
                gather_start(nb)
            if not first:
                scatter_wait(fb)
            if not last:
                idx_start(jnp.minimum(i + 2, E_CT - 1), fb)
            if last:
                scatter_wait(b)

        idx_start(0, 0)
        idx_start(1, 1)
        idx_wait(0)
        gather_start(0)
        process(jnp.int32(0), 0, first=True, last=False)
        process(jnp.int32(1), 1, first=False, last=False)
        process(jnp.int32(2), 2, first=False, last=False)

        @pl.loop(1, E_CT // NB - 1)
        def _(t):
            i = t * NB
            process(i, 0, first=False, last=False)
            process(i + 1, 1, first=False, last=False)
            process(i + 2, 2, first=False, last=False)

        process(jnp.int32(E_CT - 3), 0, first=False, last=False)
        process(jnp.int32(E_CT - 2), 1, first=False, last=False)
        process(jnp.int32(E_CT - 1), 2, first=False, last=True)
        idx_wait(E_CT % NB)
        plsc.subcore_barrier()

    hop(base3_hbm.at[c], acc1)
    for p in range(5):
        sl = pl.ds(s * RPT + p * (RPT // 5), RPT // 5)
        pltpu.sync_copy(acc1.at[sl, :], stage)
        pltpu.sync_copy(stage, mid3_hbm.at[c].at[sl, :])
    plsc.subcore_barrier()
    hop(mid3_hbm.at[c], acc2)
    for p in range(5):
        sl = pl.ds(s * RPT + p * (RPT // 5), RPT // 5)
        pltpu.sync_copy(acc2.at[sl, :], stage)
        pltpu.sync_copy(stage, out3_hbm.at[c].at[sl, :])


def kernel(adj_row, adj_col, adj_values, feat_row, feat_col, feat_values,
           W, bias):
    adj_row = adj_row.astype(jnp.int32)
    adj_col = adj_col.astype(jnp.int32)
    feat_row = feat_row.astype(jnp.int32)
    feat_col = feat_col.astype(jnp.int32)
    npad = NNZ_PAD - NNZ
    fr = jnp.concatenate([feat_row, jnp.zeros((npad,), jnp.int32)])
    fc = jnp.concatenate([feat_col, jnp.zeros((npad,), jnp.int32)])
    fv = jnp.concatenate([feat_values, jnp.zeros((npad,), jnp.float32)])
    epad = E_PAD - E
    ar = jnp.concatenate([adj_row, jnp.zeros((epad,), jnp.int32)])
    ac = jnp.concatenate([adj_col, jnp.zeros((epad,), jnp.int32)])
    av = jnp.concatenate([adj_values, jnp.zeros((epad,), jnp.float32)])

    f_flat = _fbuild(fr, fc, fv)
    base3 = _dense(f_flat.reshape(NC, N, H), W.reshape(NC, H, OUT_C), bias)
    _, out3 = _prop(ar, ac, av, base3)
    return out3.transpose(1, 0, 2).reshape(N, OUT_C)


# speedup vs baseline: 1.1878x; 1.1878x over previous
import functools

import jax
import jax.numpy as jnp
from jax import lax
from jax.experimental import pallas as pl
from jax.experimental.pallas import tpu as pltpu
from jax.experimental.pallas import tpu_sc as plsc

N = 10000
E = 320000
NNZ = 500000
IN_C = 128
OUT_C = 128
H = 64
NC = 2
NS = 16
L = 16
NB = 3

CHUNK = 128
NNZ_CT = 249
NNZ_PAD = NNZ_CT * NS * CHUNK
E_CT = 162
E_PAD = E_CT * NS * CHUNK
ACC_PAD = N * H + 512
ZSLICE = ACC_PAD // NS
OSLICE = N * H // NS
RPT = N // NS

_mesh = plsc.VectorSubcoreMesh(core_axis_name="c", subcore_axis_name="s")
_sc_params = pltpu.CompilerParams(use_tc_tiling_on_sc=False,
                                  needs_layout_passes=False)


@functools.partial(
    pl.kernel,
    out_type=jax.ShapeDtypeStruct((NC * N * H,), jnp.float32),
    mesh=_mesh,
    scratch_types=(
        [pltpu.VMEM((CHUNK,), jnp.int32) for _ in range(NB)]
        + [pltpu.VMEM((CHUNK,), jnp.int32) for _ in range(NB)]
        + [pltpu.VMEM((CHUNK,), jnp.float32) for _ in range(NB)]
        + [pltpu.VMEM((CHUNK,), jnp.int32) for _ in range(NB)]
        + [pltpu.VMEM((ZSLICE,), jnp.float32)]
        + [pltpu.VMEM_SHARED((ACC_PAD,), jnp.float32)]
        + [pltpu.SemaphoreType.DMA for _ in range(2 * NB)]
    ),
    compiler_params=_sc_params,
)
def _fbuild(rows_hbm, cols_hbm, vals_hbm, f_hbm, *refs):
    rowv = refs[0:NB]
    colv = refs[NB:2 * NB]
    valv = refs[2 * NB:3 * NB]
    idxv = refs[3 * NB:4 * NB]
    stage = refs[4 * NB]
    acc = refs[4 * NB + 1]
    isem = refs[4 * NB + 2:4 * NB + 2 + NB]
    ssem = refs[4 * NB + 2 + NB:4 * NB + 2 + 2 * NB]

    c = lax.axis_index("c")
    s = lax.axis_index("s")

    @pl.loop(0, ZSLICE, step=L)
    def _(i):
        stage[pl.ds(i, L)] = jnp.zeros((L,), jnp.float32)

    pltpu.sync_copy(stage, acc.at[pl.ds(s * ZSLICE, ZSLICE)])
    plsc.subcore_barrier()
    col_off = c * H

    def idx_start(i, b):
        base = (s + i * NS) * CHUNK
        pltpu.async_copy(rows_hbm.at[pl.ds(base, CHUNK)], rowv[b], isem[b])
        pltpu.async_copy(cols_hbm.at[pl.ds(base, CHUNK)], colv[b], isem[b])
        pltpu.async_copy(vals_hbm.at[pl.ds(base, CHUNK)], valv[b], isem[b])

    def idx_wait(b):
        pltpu.make_async_copy(rows_hbm.at[pl.ds(0, CHUNK)], rowv[b], isem[b]).wait()
        pltpu.make_async_copy(cols_hbm.at[pl.ds(0, CHUNK)], colv[b], isem[b]).wait()
        pltpu.make_async_copy(vals_hbm.at[pl.ds(0, CHUNK)], valv[b], isem[b]).wait()

    def compute(b):
        for j in range(CHUNK // L):
            sl = pl.ds(j * L, L)
            r = rowv[b][sl]
            cc = colv[b][sl] - col_off
            ok = (cc >= 0) & (cc < H)
            idxv[b][sl] = jnp.where(ok, r * H + cc, N * H)

    def scatter_start(b):
        pltpu.async_copy(valv[b], acc.at[idxv[b]], ssem[b], add=True)

    def scatter_wait(b):
        pltpu.make_async_copy(valv[b], acc.at[idxv[b]], ssem[b]).wait()

    def process(i, b, first, last):
        fb = (b + 2) % NB
        idx_wait(b)
        compute(b)
        scatter_start(b)
        if not first:
            scatter_wait(fb)
        if not last:
            idx_start(jnp.minimum(i + 2, NNZ_CT - 1), fb)

    idx_start(0, 0)
    idx_start(1, 1)
    process(jnp.int32(0), 0, first=True, last=False)
    process(jnp.int32(1), 1, first=False, last=False)
    process(jnp.int32(2), 2, first=False, last=False)

    @pl.loop(1, NNZ_CT // NB - 1)
    def _(t):
        i = t * NB
        process(i, 0, first=False, last=False)
        process(i + 1, 1, first=False, last=False)
        process(i + 2, 2, first=False, last=False)

    process(jnp.int32(NNZ_CT - 3), 0, first=False, last=False)
    process(jnp.int32(NNZ_CT - 2), 1, first=False, last=False)
    process(jnp.int32(NNZ_CT - 1), 2, first=False, last=True)
    scatter_wait(2)
    idx_wait(NNZ_CT % NB)

    plsc.subcore_barrier()
    pltpu.sync_copy(acc.at[pl.ds(s * OSLICE, OSLICE)], stage.at[pl.ds(0, OSLICE)])
    pltpu.sync_copy(stage.at[pl.ds(0, OSLICE)],
                    f_hbm.at[pl.ds(c * N * H + s * OSLICE, OSLICE)])


def _dense_body(f_ref, w_ref, b_ref, o_ref):
    f = f_ref[...]
    o = jnp.dot(f[0], w_ref[0], preferred_element_type=jnp.float32)
    o = o + jnp.dot(f[1], w_ref[1], preferred_element_type=jnp.float32)
    o = jnp.maximum(o + b_ref[...], 0.0)
    o_ref[...] = jnp.stack([o[:, :H], o[:, H:]])


_BM = 2000


def _dense(f3, w3, bias):
    return pl.pallas_call(
        _dense_body,
        grid=(N // _BM,),
        in_specs=[
            pl.BlockSpec((NC, _BM, H), lambda i: (0, i, 0)),
            pl.BlockSpec((NC, H, OUT_C), lambda i: (0, 0, 0)),
            pl.BlockSpec((1, OUT_C), lambda i: (0, 0)),
        ],
        out_specs=pl.BlockSpec((NC, _BM, H), lambda i: (0, i, 0)),
        out_shape=jax.ShapeDtypeStruct((NC, N, H), jnp.float32),
    )(f3, w3, bias)


@functools.partial(
    pl.kernel,
    out_type=[jax.ShapeDtypeStruct((NC, N, H), jnp.float32),
              jax.ShapeDtypeStruct((NC, N, H), jnp.float32)],
    mesh=_mesh,
    scratch_types=(
        [pltpu.VMEM((CHUNK,), jnp.int32) for _ in range(NB)]
        + [pltpu.VMEM((CHUNK,), jnp.int32) for _ in range(NB)]
        + [pltpu.VMEM((CHUNK,), jnp.float32) for _ in range(NB)]
        + [pltpu.VMEM((CHUNK, H), jnp.float32) for _ in range(NB)]
        + [pltpu.VMEM((RPT // 5, H), jnp.float32)]
        + [pltpu.VMEM_SHARED((N, H), jnp.float32)]
        + [pltpu.VMEM_SHARED((N, H), jnp.float32)]
        + [pltpu.SemaphoreType.DMA for _ in range(3 * NB)]
    ),
    compiler_params=_sc_params,
)
def _prop(adjr_hbm, adjc_hbm, adjv_hbm, base3_hbm, mid3_hbm, out3_hbm, *refs):
    rowv = refs[0:NB]
    colv = refs[NB:2 * NB]
    valv = refs[2 * NB:3 * NB]
    rbuf = refs[3 * NB:4 * NB]
    stage = refs[4 * NB]
    acc1 = refs[4 * NB + 1]
    acc2 = refs[4 * NB + 2]
    isem = refs[4 * NB + 3:4 * NB + 3 + NB]
    gsem = refs[4 * NB + 3 + NB:4 * NB + 3 + 2 * NB]
    ssem = refs[4 * NB + 3 + 2 * NB:4 * NB + 3 + 3 * NB]

    c = lax.axis_index("c")
    s = lax.axis_index("s")

    @pl.loop(0, RPT // 5)
    def _(k):
        for j in range(H // L):
            stage[k, pl.ds(j * L, L)] = jnp.zeros((L,), jnp.float32)

    for p in range(5):
        pltpu.sync_copy(stage, acc1.at[pl.ds(s * RPT + p * (RPT // 5), RPT // 5), :])
        pltpu.sync_copy(stage, acc2.at[pl.ds(s * RPT + p * (RPT // 5), RPT // 5), :])
    plsc.subcore_barrier()

    def idx_start(i, b):
        base = (s + i * NS) * CHUNK
        pltpu.async_copy(adjc_hbm.at[pl.ds(base, CHUNK)], colv[b], isem[b])
        pltpu.async_copy(adjr_hbm.at[pl.ds(base, CHUNK)], rowv[b], isem[b])
        pltpu.async_copy(adjv_hbm.at[pl.ds(base, CHUNK)], valv[b], isem[b])

    def idx_wait(b):
        pltpu.make_async_copy(adjc_hbm.at[pl.ds(0, CHUNK)], colv[b], isem[b]).wait()
        pltpu.make_async_copy(adjr_hbm.at[pl.ds(0, CHUNK)], rowv[b], isem[b]).wait()
        pltpu.make_async_copy(adjv_hbm.at[pl.ds(0, CHUNK)], valv[b], isem[b]).wait()

    def compute(b):
        @pl.loop(0, CHUNK, step=L)
        def _(k0):
            for e in range(L):
                v = plsc.load_gather(valv[b], [jnp.full((L,), k0 + e, jnp.int32)])
                for j in range(H // L):
                    sl = pl.ds(j * L, L)
                    rbuf[b][k0 + e, sl] = rbuf[b][k0 + e, sl] * v

    def hop(src, dst_acc):
        def gather_start(b):
            pltpu.async_copy(src.at[colv[b]], rbuf[b], gsem[b])

        def gather_wait(b):
            pltpu.make_async_copy(src.at[colv[b]], rbuf[b], gsem[b]).wait()

        def scatter_start(b):
            pltpu.async_copy(rbuf[b], dst_acc.at[rowv[b]], ssem[b], add=True)

        def scatter_wait(b):
            pltpu.make_async_copy(rbuf[b], dst_acc.at[rowv[b]], ssem[b]).wait()

        def process(i, b, first, last):
            nb = (b + 1) % NB
            fb = (b + 2) % NB
            gather_wait(b)
            compute(b)
            scatter_start(b)
            if not last:
                idx_wait(nb)
                gather_start(nb)
            if not first:
                scatter_wait(fb)
            if not last:
                idx_start(jnp.minimum(i + 2, E_CT - 1), fb)
            if last:
                scatter_wait(b)

        idx_start(0, 0)
        idx_start(1, 1)
        idx_wait(0)
        gather_start(0)
        process(jnp.int32(0), 0, first=True, last=False)
        process(jnp.int32(1), 1, first=False, last=False)
        process(jnp.int32(2), 2, first=False, last=False)

        @pl.loop(1, E_CT // NB - 1)
        def _(t):
            i = t * NB
            process(i, 0, first=False, last=False)
            process(i + 1, 1, first=False, last=False)
            process(i + 2, 2, first=False, last=False)

        process(jnp.int32(E_CT - 3), 0, first=False, last=False)
        process(jnp.int32(E_CT - 2), 1, first=False, last=False)
        process(jnp.int32(E_CT - 1), 2, first=False, last=True)
        idx_wait(E_CT % NB)
        plsc.subcore_barrier()

    hop(base3_hbm.at[c], acc1)
    hop(acc1, acc2)
    for p in range(5):
        sl = pl.ds(s * RPT + p * (RPT // 5), RPT // 5)
        pltpu.sync_copy(acc1.at[sl, :], stage)
        pltpu.sync_copy(stage, mid3_hbm.at[c].at[sl, :])
    for p in range(5):
        sl = pl.ds(s * RPT + p * (RPT // 5), RPT // 5)
        pltpu.sync_copy(acc2.at[sl, :], stage)
        pltpu.sync_copy(stage, out3_hbm.at[c].at[sl, :])


def kernel(adj_row, adj_col, adj_values, feat_row, feat_col, feat_values,
           W, bias):
    adj_row = adj_row.astype(jnp.int32)
    adj_col = adj_col.astype(jnp.int32)
    feat_row = feat_row.astype(jnp.int32)
    feat_col = feat_col.astype(jnp.int32)
    npad = NNZ_PAD - NNZ
    fr = jnp.concatenate([feat_row, jnp.zeros((npad,), jnp.int32)])
    fc = jnp.concatenate([feat_col, jnp.zeros((npad,), jnp.int32)])
    fv = jnp.concatenate([feat_values, jnp.zeros((npad,), jnp.float32)])
    epad = E_PAD - E
    ar = jnp.concatenate([adj_row, jnp.zeros((epad,), jnp.int32)])
    ac = jnp.concatenate([adj_col, jnp.zeros((epad,), jnp.int32)])
    av = jnp.concatenate([adj_values, jnp.zeros((epad,), jnp.float32)])

    f_flat = _fbuild(fr, fc, fv)
    base3 = _dense(f_flat.reshape(NC, N, H), W.reshape(NC, H, OUT_C), bias)
    _, out3 = _prop(ar, ac, av, base3)
    return out3.transpose(1, 0, 2).reshape(N, OUT_C)


# speedup vs baseline: 1.9914x; 1.6765x over previous
import functools

import jax
import jax.numpy as jnp
from jax import lax
from jax.experimental import pallas as pl
from jax.experimental.pallas import tpu as pltpu
from jax.experimental.pallas import tpu_sc as plsc

N = 10000
E = 320000
NNZ = 500000
IN_C = 128
OUT_C = 128
H = 64
NC = 2
NS = 16
L = 16
NB = 3

CHUNK = 128
NNZ_CT = 249
NNZ_PAD = NNZ_CT * NS * CHUNK
E_CT = 162
E_PAD = E_CT * NS * CHUNK
ACC_PAD = N * H + 512
ZSLICE = ACC_PAD // NS
OSLICE = N * H // NS
RPT = N // NS

_mesh = plsc.VectorSubcoreMesh(core_axis_name="c", subcore_axis_name="s")
_sc_params = pltpu.CompilerParams(use_tc_tiling_on_sc=False,
                                  needs_layout_passes=False)


@functools.partial(
    pl.kernel,
    out_type=jax.ShapeDtypeStruct((NC * N * H,), jnp.float32),
    mesh=_mesh,
    scratch_types=(
        [pltpu.VMEM((CHUNK,), jnp.int32) for _ in range(NB)]
        + [pltpu.VMEM((CHUNK,), jnp.int32) for _ in range(NB)]
        + [pltpu.VMEM((CHUNK,), jnp.float32) for _ in range(NB)]
        + [pltpu.VMEM((CHUNK,), jnp.int32) for _ in range(NB)]
        + [pltpu.VMEM((ZSLICE,), jnp.float32)]
        + [pltpu.VMEM_SHARED((ACC_PAD,), jnp.float32)]
        + [pltpu.SemaphoreType.DMA for _ in range(2 * NB)]
    ),
    compiler_params=_sc_params,
)
def _fbuild(rows_hbm, cols_hbm, vals_hbm, f_hbm, *refs):
    rowv = refs[0:NB]
    colv = refs[NB:2 * NB]
    valv = refs[2 * NB:3 * NB]
    idxv = refs[3 * NB:4 * NB]
    stage = refs[4 * NB]
    acc = refs[4 * NB + 1]
    isem = refs[4 * NB + 2:4 * NB + 2 + NB]
    ssem = refs[4 * NB + 2 + NB:4 * NB + 2 + 2 * NB]

    c = lax.axis_index("c")
    s = lax.axis_index("s")

    @pl.loop(0, ZSLICE, step=L)
    def _(i):
        stage[pl.ds(i, L)] = jnp.zeros((L,), jnp.float32)

    pltpu.sync_copy(stage, acc.at[pl.ds(s * ZSLICE, ZSLICE)])
    plsc.subcore_barrier()
    col_off = c * H

    def idx_start(i, b):
        base = (s + i * NS) * CHUNK
        pltpu.async_copy(rows_hbm.at[pl.ds(base, CHUNK)], rowv[b], isem[b])
        pltpu.async_copy(cols_hbm.at[pl.ds(base, CHUNK)], colv[b], isem[b])
        pltpu.async_copy(vals_hbm.at[pl.ds(base, CHUNK)], valv[b], isem[b])

    def idx_wait(b):
        pltpu.make_async_copy(rows_hbm.at[pl.ds(0, CHUNK)], rowv[b], isem[b]).wait()
        pltpu.make_async_copy(cols_hbm.at[pl.ds(0, CHUNK)], colv[b], isem[b]).wait()
        pltpu.make_async_copy(vals_hbm.at[pl.ds(0, CHUNK)], valv[b], isem[b]).wait()

    def compute(b):
        for j in range(CHUNK // L):
            sl = pl.ds(j * L, L)
            r = rowv[b][sl]
            cc = colv[b][sl] - col_off
            ok = (cc >= 0) & (cc < H)
            idxv[b][sl] = jnp.where(ok, r * H + cc, N * H)

    def scatter_start(b):
        pltpu.async_copy(valv[b], acc.at[idxv[b]], ssem[b], add=True)

    def scatter_wait(b):
        pltpu.make_async_copy(valv[b], acc.at[idxv[b]], ssem[b]).wait()

    def process(i, b, first, last):
        fb = (b + 2) % NB
        idx_wait(b)
        compute(b)
        scatter_start(b)
        if not first:
            scatter_wait(fb)
        if not last:
            idx_start(jnp.minimum(i + 2, NNZ_CT - 1), fb)

    idx_start(0, 0)
    idx_start(1, 1)
    process(jnp.int32(0), 0, first=True, last=False)

    @pl.loop(1, NNZ_CT - 1)
    def _(i):
        b = lax.rem(i, NB)
        for bb in range(NB):
            @pl.when(b == bb)
            def _():
                process(i, bb, first=False, last=False)

    process(jnp.int32(NNZ_CT - 1), (NNZ_CT - 1) % NB, first=False, last=True)
    scatter_wait((NNZ_CT - 1) % NB)
    idx_wait(NNZ_CT % NB)

    plsc.subcore_barrier()
    pltpu.sync_copy(acc.at[pl.ds(s * OSLICE, OSLICE)], stage.at[pl.ds(0, OSLICE)])
    pltpu.sync_copy(stage.at[pl.ds(0, OSLICE)],
                    f_hbm.at[pl.ds(c * N * H + s * OSLICE, OSLICE)])


def _dense_body(f_ref, w_ref, b_ref, o_ref):
    f = f_ref[...]
    o = jnp.dot(f[0], w_ref[0], preferred_element_type=jnp.float32)
    o = o + jnp.dot(f[1], w_ref[1], preferred_element_type=jnp.float32)
    o = jnp.maximum(o + b_ref[...], 0.0)
    o_ref[...] = jnp.stack([o[:, :H], o[:, H:]]).astype(jnp.bfloat16)


_BM = 2000


def _dense(f3, w3, bias):
    return pl.pallas_call(
        _dense_body,
        grid=(N // _BM,),
        in_specs=[
            pl.BlockSpec((NC, _BM, H), lambda i: (0, i, 0)),
            pl.BlockSpec((NC, H, OUT_C), lambda i: (0, 0, 0)),
            pl.BlockSpec((1, OUT_C), lambda i: (0, 0)),
        ],
        out_specs=pl.BlockSpec((NC, _BM, H), lambda i: (0, i, 0)),
        out_shape=jax.ShapeDtypeStruct((NC, N, H), jnp.bfloat16),
    )(f3, w3, bias)


@functools.partial(
    pl.kernel,
    out_type=jax.ShapeDtypeStruct((NC, N, H), jnp.bfloat16),
    mesh=_mesh,
    scratch_types=(
        [pltpu.VMEM((CHUNK,), jnp.int32) for _ in range(NB)]
        + [pltpu.VMEM((CHUNK,), jnp.int32) for _ in range(NB)]
        + [pltpu.VMEM((CHUNK,), jnp.float32) for _ in range(NB)]
        + [pltpu.VMEM((CHUNK, H), jnp.bfloat16) for _ in range(NB)]
        + [pltpu.VMEM((RPT // 5, H), jnp.bfloat16)]
        + [pltpu.VMEM_SHARED((N, H), jnp.bfloat16)]
        + [pltpu.VMEM_SHARED((N, H), jnp.bfloat16)]
        + [pltpu.SemaphoreType.DMA for _ in range(3 * NB)]
    ),
    compiler_params=_sc_params,
)
def _prop(adjr_hbm, adjc_hbm, adjv_hbm, base3_hbm, out3_hbm, *refs):
    rowv = refs[0:NB]
    colv = refs[NB:2 * NB]
    valv = refs[2 * NB:3 * NB]
    rbuf = refs[3 * NB:4 * NB]
    stage = refs[4 * NB]
    acc1 = refs[4 * NB + 1]
    acc2 = refs[4 * NB + 2]
    isem = refs[4 * NB + 3:4 * NB + 3 + NB]
    gsem = refs[4 * NB + 3 + NB:4 * NB + 3 + 2 * NB]
    ssem = refs[4 * NB + 3 + 2 * NB:4 * NB + 3 + 3 * NB]

    c = lax.axis_index("c")
    s = lax.axis_index("s")

    @pl.loop(0, RPT // 5)
    def _(k):
        for j in range(H // (2 * L)):
            stage[k, pl.ds(j * 2 * L, 2 * L)] = jnp.zeros((2 * L,), jnp.bfloat16)

    for p in range(5):
        pltpu.sync_copy(stage, acc1.at[pl.ds(s * RPT + p * (RPT // 5), RPT // 5), :])
        pltpu.sync_copy(stage, acc2.at[pl.ds(s * RPT + p * (RPT // 5), RPT // 5), :])
    plsc.subcore_barrier()

    def idx_start(i, b):
        base = (s + i * NS) * CHUNK
        pltpu.async_copy(adjc_hbm.at[pl.ds(base, CHUNK)], colv[b], isem[b])
        pltpu.async_copy(adjr_hbm.at[pl.ds(base, CHUNK)], rowv[b], isem[b])
        pltpu.async_copy(adjv_hbm.at[pl.ds(base, CHUNK)], valv[b], isem[b])

    def idx_wait(b):
        pltpu.make_async_copy(adjc_hbm.at[pl.ds(0, CHUNK)], colv[b], isem[b]).wait()
        pltpu.make_async_copy(adjr_hbm.at[pl.ds(0, CHUNK)], rowv[b], isem[b]).wait()
        pltpu.make_async_copy(adjv_hbm.at[pl.ds(0, CHUNK)], valv[b], isem[b]).wait()

    def compute(b):
        @pl.loop(0, CHUNK, step=L)
        def _(k0):
            for e in range(L):
                v = plsc.load_gather(valv[b], [jnp.full((L,), k0 + e, jnp.int32)])
                vb = plsc.pack(v, v, format=plsc.PackFormat.INTERLEAVED)
                for j in range(H // (2 * L)):
                    sl = pl.ds(j * 2 * L, 2 * L)
                    rbuf[b][k0 + e, sl] = rbuf[b][k0 + e, sl] * vb

    def hop(src, dst_acc):
        def gather_start(b):
            pltpu.async_copy(src.at[colv[b]], rbuf[b], gsem[b])

        def gather_wait(b):
            pltpu.make_async_copy(src.at[colv[b]], rbuf[b], gsem[b]).wait()

        def scatter_start(b):
            pltpu.async_copy(rbuf[b], dst_acc.at[rowv[b]], ssem[b], add=True)

        def scatter_wait(b):
            pltpu.make_async_copy(rbuf[b], dst_acc.at[rowv[b]], ssem[b]).wait()

        def process(i, b, first, last):
            nb = (b + 1) % NB
            fb = (b + 2) % NB
            gather_wait(b)
            compute(b)
            scatter_start(b)
            if not last:
                idx_wait(nb)
                gather_start(nb)
            if not first:
                scatter_wait(fb)
            if not last:
                idx_start(jnp.minimum(i + 2, E_CT - 1), fb)
            if last:
                scatter_wait(b)

        idx_start(0, 0)
        idx_start(1, 1)
        idx_wait(0)
        gather_start(0)
        process(jnp.int32(0), 0, first=True, last=False)

        @pl.loop(1, E_CT - 1)
        def _(i):
            b = lax.rem(i, NB)
            for bb in range(NB):
                @pl.when(b == bb)
                def _():
                    process(i, bb, first=False, last=False)

        process(jnp.int32(E_CT - 1), (E_CT - 1) % NB, first=False, last=True)
        idx_wait(E_CT % NB)
        plsc.subcore_barrier()

    hop(base3_hbm.at[c], acc1)
    hop(acc1, acc2)
    for p in range(5):
        sl = pl.ds(s * RPT + p * (RPT // 5), RPT // 5)
        pltpu.sync_copy(acc2.at[sl, :], stage)
        pltpu.sync_copy(stage, out3_hbm.at[c].at[sl, :])


def kernel(adj_row, adj_col, adj_values, feat_row, feat_col, feat_values,
           W, bias):
    adj_row = adj_row.astype(jnp.int32)
    adj_col = adj_col.astype(jnp.int32)
    feat_row = feat_row.astype(jnp.int32)
    feat_col = feat_col.astype(jnp.int32)
    npad = NNZ_PAD - NNZ
    fr = jnp.concatenate([feat_row, jnp.zeros((npad,), jnp.int32)])
    fc = jnp.concatenate([feat_col, jnp.zeros((npad,), jnp.int32)])
    fv = jnp.concatenate([feat_values, jnp.zeros((npad,), jnp.float32)])
    epad = E_PAD - E
    ar = jnp.concatenate([adj_row, jnp.zeros((epad,), jnp.int32)])
    ac = jnp.concatenate([adj_col, jnp.zeros((epad,), jnp.int32)])
    av = jnp.concatenate([adj_values, jnp.zeros((epad,), jnp.float32)])

    f_flat = _fbuild(fr, fc, fv)
    base3 = _dense(f_flat.reshape(NC, N, H), W.reshape(NC, H, OUT_C), bias)
    out3 = _prop(ar, ac, av, base3)
    return out3.transpose(1, 0, 2).reshape(N, OUT_C).astype(jnp.float32)


# speedup vs baseline: 2.8015x; 1.4068x over previous
import functools

import jax
import jax.numpy as jnp
from jax import lax
from jax.experimental import pallas as pl
from jax.experimental.pallas import tpu as pltpu
from jax.experimental.pallas import tpu_sc as plsc

N = 10000
E = 320000
NNZ = 500000
IN_C = 128
OUT_C = 128
H = 64
NC = 2
NS = 16
L = 16
NB = 3

CHUNK = 128
NNZ_CT = 123
NNZ_PAD = NNZ_CT * NC * NS * CHUNK
E_CT = 162
E_PAD = E_CT * NS * CHUNK
ACC_PAD = N * IN_C + 1024
ZSLICE = ACC_PAD // NS
OSLICE = N * IN_C // NS
QS = 20016
RPT = N // NS

_mesh = plsc.VectorSubcoreMesh(core_axis_name="c", subcore_axis_name="s")
_sc_params = pltpu.CompilerParams(use_tc_tiling_on_sc=False,
                                  needs_layout_passes=False)


@functools.partial(
    pl.kernel,
    out_type=jax.ShapeDtypeStruct((NC * N * IN_C,), jnp.float32),
    mesh=_mesh,
    scratch_types=(
        [pltpu.VMEM((CHUNK,), jnp.int32) for _ in range(NB)]
        + [pltpu.VMEM((CHUNK,), jnp.int32) for _ in range(NB)]
        + [pltpu.VMEM((CHUNK,), jnp.float32) for _ in range(NB)]
        + [pltpu.VMEM((CHUNK,), jnp.int32) for _ in range(NB)]
        + [pltpu.VMEM((QS,), jnp.float32)]
        + [pltpu.VMEM_SHARED((ACC_PAD,), jnp.float32)]
        + [pltpu.SemaphoreType.DMA for _ in range(2 * NB)]
    ),
    compiler_params=_sc_params,
)
def _fbuild(rows_hbm, cols_hbm, vals_hbm, f_hbm, *refs):
    rowv = refs[0:NB]
    colv = refs[NB:2 * NB]
    valv = refs[2 * NB:3 * NB]
    idxv = refs[3 * NB:4 * NB]
    stage = refs[4 * NB]
    acc = refs[4 * NB + 1]
    isem = refs[4 * NB + 2:4 * NB + 2 + NB]
    ssem = refs[4 * NB + 2 + NB:4 * NB + 2 + 2 * NB]

    c = lax.axis_index("c")
    s = lax.axis_index("s")

    @pl.loop(0, QS, step=L)
    def _(i):
        stage[pl.ds(i, L)] = jnp.zeros((L,), jnp.float32)

    for p in range(ZSLICE // QS):
        pltpu.sync_copy(stage, acc.at[pl.ds(s * ZSLICE + p * QS, QS)])
    plsc.subcore_barrier()

    def idx_start(i, b):
        base = (c * (NS * NNZ_CT) + s + i * NS) * CHUNK
        pltpu.async_copy(rows_hbm.at[pl.ds(base, CHUNK)], rowv[b], isem[b])
        pltpu.async_copy(cols_hbm.at[pl.ds(base, CHUNK)], colv[b], isem[b])
        pltpu.async_copy(vals_hbm.at[pl.ds(base, CHUNK)], valv[b], isem[b])

    def idx_wait(b):
        pltpu.make_async_copy(rows_hbm.at[pl.ds(0, CHUNK)], rowv[b], isem[b]).wait()
        pltpu.make_async_copy(cols_hbm.at[pl.ds(0, CHUNK)], colv[b], isem[b]).wait()
        pltpu.make_async_copy(vals_hbm.at[pl.ds(0, CHUNK)], valv[b], isem[b]).wait()

    def compute(b):
        for j in range(CHUNK // L):
            sl = pl.ds(j * L, L)
            idxv[b][sl] = rowv[b][sl] * IN_C + colv[b][sl]

    def scatter_start(b):
        pltpu.async_copy(valv[b], acc.at[idxv[b]], ssem[b], add=True)

    def scatter_wait(b):
        pltpu.make_async_copy(valv[b], acc.at[idxv[b]], ssem[b]).wait()

    def process(i, b, first, last):
        fb = (b + 2) % NB
        idx_wait(b)
        compute(b)
        scatter_start(b)
        if not first:
            scatter_wait(fb)
        if not last:
            idx_start(jnp.minimum(i + 2, NNZ_CT - 1), fb)

    idx_start(0, 0)
    idx_start(1, 1)
    process(jnp.int32(0), 0, first=True, last=False)

    @pl.loop(1, NNZ_CT - 1)
    def _(i):
        b = lax.rem(i, NB)
        for bb in range(NB):
            @pl.when(b == bb)
            def _():
                process(i, bb, first=False, last=False)

    process(jnp.int32(NNZ_CT - 1), (NNZ_CT - 1) % NB, first=False, last=True)
    scatter_wait((NNZ_CT - 1) % NB)
    idx_wait(NNZ_CT % NB)

    plsc.subcore_barrier()
    for p in range(4):
        ofs = s * OSLICE + p * (OSLICE // 4)
        pltpu.sync_copy(acc.at[pl.ds(ofs, OSLICE // 4)],
                        stage.at[pl.ds(0, OSLICE // 4)])
        pltpu.sync_copy(stage.at[pl.ds(0, OSLICE // 4)],
                        f_hbm.at[pl.ds(c * (N * IN_C) + ofs, OSLICE // 4)])


def _dense_body(f_ref, w_ref, b_ref, o_ref):
    f = f_ref[0] + f_ref[1]
    o = jnp.dot(f, w_ref[...], preferred_element_type=jnp.float32)
    o = jnp.maximum(o + b_ref[...], 0.0)
    o_ref[...] = jnp.stack([o[:, :H], o[:, H:]]).astype(jnp.bfloat16)


_BM = 2000


def _dense(f3, w, bias):
    return pl.pallas_call(
        _dense_body,
        grid=(N // _BM,),
        in_specs=[
            pl.BlockSpec((NC, _BM, IN_C), lambda i: (0, i, 0)),
            pl.BlockSpec((IN_C, OUT_C), lambda i: (0, 0)),
            pl.BlockSpec((1, OUT_C), lambda i: (0, 0)),
        ],
        out_specs=pl.BlockSpec((NC, _BM, H), lambda i: (0, i, 0)),
        out_shape=jax.ShapeDtypeStruct((NC, N, H), jnp.bfloat16),
    )(f3, w, bias)


@functools.partial(
    pl.kernel,
    out_type=jax.ShapeDtypeStruct((NC, N, H), jnp.bfloat16),
    mesh=_mesh,
    scratch_types=(
        [pltpu.VMEM((CHUNK,), jnp.int32) for _ in range(NB)]
        + [pltpu.VMEM((CHUNK,), jnp.int32) for _ in range(NB)]
        + [pltpu.VMEM((CHUNK,), jnp.float32) for _ in range(NB)]
        + [pltpu.VMEM((CHUNK, H), jnp.bfloat16) for _ in range(NB)]
        + [pltpu.VMEM((RPT // 5, H), jnp.bfloat16)]
        + [pltpu.VMEM_SHARED((N, H), jnp.bfloat16)]
        + [pltpu.VMEM_SHARED((N, H), jnp.bfloat16)]
        + [pltpu.SemaphoreType.DMA for _ in range(3 * NB)]
    ),
    compiler_params=_sc_params,
)
def _prop(adjr_hbm, adjc_hbm, adjv_hbm, base3_hbm, out3_hbm, *refs):
    rowv = refs[0:NB]
    colv = refs[NB:2 * NB]
    valv = refs[2 * NB:3 * NB]
    rbuf = refs[3 * NB:4 * NB]
    stage = refs[4 * NB]
    acc1 = refs[4 * NB + 1]
    acc2 = refs[4 * NB + 2]
    isem = refs[4 * NB + 3:4 * NB + 3 + NB]
    gsem = refs[4 * NB + 3 + NB:4 * NB + 3 + 2 * NB]
    ssem = refs[4 * NB + 3 + 2 * NB:4 * NB + 3 + 3 * NB]

    c = lax.axis_index("c")
    s = lax.axis_index("s")

    @pl.loop(0, RPT // 5)
    def _(k):
        for j in range(H // (2 * L)):
            stage[k, pl.ds(j * 2 * L, 2 * L)] = jnp.zeros((2 * L,), jnp.bfloat16)

    for p in range(5):
        pltpu.sync_copy(stage, acc1.at[pl.ds(s * RPT + p * (RPT // 5), RPT // 5), :])
        pltpu.sync_copy(stage, acc2.at[pl.ds(s * RPT + p * (RPT // 5), RPT // 5), :])
    plsc.subcore_barrier()

    def idx_start(i, b):
        base = (s + i * NS) * CHUNK
        pltpu.async_copy(adjc_hbm.at[pl.ds(base, CHUNK)], colv[b], isem[b])
        pltpu.async_copy(adjr_hbm.at[pl.ds(base, CHUNK)], rowv[b], isem[b])
        pltpu.async_copy(adjv_hbm.at[pl.ds(base, CHUNK)], valv[b], isem[b])

    def idx_wait(b):
        pltpu.make_async_copy(adjc_hbm.at[pl.ds(0, CHUNK)], colv[b], isem[b]).wait()
        pltpu.make_async_copy(adjr_hbm.at[pl.ds(0, CHUNK)], rowv[b], isem[b]).wait()
        pltpu.make_async_copy(adjv_hbm.at[pl.ds(0, CHUNK)], valv[b], isem[b]).wait()

    def compute(b):
        @pl.loop(0, CHUNK, step=L)
        def _(k0):
            for e in range(L):
                v = plsc.load_gather(valv[b], [jnp.full((L,), k0 + e, jnp.int32)])
                vb = plsc.pack(v, v, format=plsc.PackFormat.INTERLEAVED)
                for j in range(H // (2 * L)):
                    sl = pl.ds(j * 2 * L, 2 * L)
                    rbuf[b][k0 + e, sl] = rbuf[b][k0 + e, sl] * vb

    def hop(src, dst_acc):
        def gather_start(b):
            pltpu.async_copy(src.at[colv[b]], rbuf[b], gsem[b])

        def gather_wait(b):
            pltpu.make_async_copy(src.at[colv[b]], rbuf[b], gsem[b]).wait()

        def scatter_start(b):
            pltpu.async_copy(rbuf[b], dst_acc.at[rowv[b]], ssem[b], add=True)

        def scatter_wait(b):
            pltpu.make_async_copy(rbuf[b], dst_acc.at[rowv[b]], ssem[b]).wait()

        def process(i, b, first, last):
            nb = (b + 1) % NB
            fb = (b + 2) % NB
            gather_wait(b)
            compute(b)
            scatter_start(b)
            if not last:
                idx_wait(nb)
                gather_start(nb)
            if not first:
                scatter_wait(fb)
            if not last:
                idx_start(jnp.minimum(i + 2, E_CT - 1), fb)
            if last:
                scatter_wait(b)

        idx_start(0, 0)
        idx_start(1, 1)
        idx_wait(0)
        gather_start(0)
        process(jnp.int32(0), 0, first=True, last=False)

        @pl.loop(1, E_CT - 1)
        def _(i):
            b = lax.rem(i, NB)
            for bb in range(NB):
                @pl.when(b == bb)
                def _():
                    process(i, bb, first=False, last=False)

        process(jnp.int32(E_CT - 1), (E_CT - 1) % NB, first=False, last=True)
        idx_wait(E_CT % NB)
        plsc.subcore_barrier()

    hop(base3_hbm.at[c], acc1)
    hop(acc1, acc2)
    for p in range(5):
        sl = pl.ds(s * RPT + p * (RPT // 5), RPT // 5)
        pltpu.sync_copy(acc2.at[sl, :], stage)
        pltpu.sync_copy(stage, out3_hbm.at[c].at[sl, :])


def kernel(adj_row, adj_col, adj_values, feat_row, feat_col, feat_values,
           W, bias):
    adj_row = adj_row.astype(jnp.int32)
    adj_col = adj_col.astype(jnp.int32)
    feat_row = feat_row.astype(jnp.int32)
    feat_col = feat_col.astype(jnp.int32)
    npad = NNZ_PAD - NNZ
    fr = jnp.concatenate([feat_row, jnp.zeros((npad,), jnp.int32)])
    fc = jnp.concatenate([feat_col, jnp.zeros((npad,), jnp.int32)])
    fv = jnp.concatenate([feat_values, jnp.zeros((npad,), jnp.float32)])
    epad = E_PAD - E
    ar = jnp.concatenate([adj_row, jnp.zeros((epad,), jnp.int32)])
    ac = jnp.concatenate([adj_col, jnp.zeros((epad,), jnp.int32)])
    av = jnp.concatenate([adj_values, jnp.zeros((epad,), jnp.float32)])

    f_flat = _fbuild(fr, fc, fv)
    base3 = _dense(f_flat.reshape(NC, N, IN_C), W, bias)
    out3 = _prop(ar, ac, av, base3)
    return out3.transpose(1, 0, 2).reshape(N, OUT_C).astype(jnp.float32)
